# Initial kernel scaffold; baseline (speedup 1.0000x reference)
import jax
import jax.numpy as jnp
from jax.experimental import pallas as pl


def kernel(x, edge_index, batch, gin_W1, gin_b1, gin_W2, gin_b2, bn_g, bn_b, sep_W1, sep_b1, sep_bn_g, sep_bn_b, sep_W2, sep_b2):
    raise NotImplementedError("write your pallas kernel here")


# speedup vs baseline: 4.2365x; 4.2365x over previous
import functools

import jax
import jax.numpy as jnp
from jax import lax
from jax.experimental import pallas as pl
from jax.experimental.pallas import tpu as pltpu
from jax.experimental.pallas import tpu_sc as plsc

_N = 10000
_E = 320000
_D = 128
_G = 128
_L = 5

_NC = 2
_NS = 16
_NW = _NC * _NS
_EPT = _E // _NW
_K = 80
_NCHUNK = _EPT // _K
_NPAD = 10240
_ROWS_PER_TILE = _NPAD // _NS

_HI = jax.lax.Precision.HIGHEST


def _segsum_body(h_hbm, src_hbm, dst_hbm, zero_hbm, out_hbm,
                 src_v, dst_v, rows_v, agg_sh, sem):
    c = lax.axis_index("c")
    s = lax.axis_index("s")

    @pl.when(s == 0)
    def _():
        pltpu.sync_copy(zero_hbm, agg_sh)

    plsc.subcore_barrier()

    tbase = (c * _NS + s) * _EPT

    def chunk(j, carry):
        base = tbase + j * _K
        pltpu.sync_copy(src_hbm.at[pl.ds(base, _K)], src_v)
        pltpu.sync_copy(dst_hbm.at[pl.ds(base, _K)], dst_v)
        pltpu.async_copy(h_hbm.at[src_v], rows_v, sem).wait()
        pltpu.sync_copy(rows_v, agg_sh.at[dst_v], add=True)
        return carry

    lax.fori_loop(0, _NCHUNK, chunk, 0)

    plsc.subcore_barrier()
    r0 = s * _ROWS_PER_TILE
    pltpu.sync_copy(agg_sh.at[pl.ds(r0, _ROWS_PER_TILE)],
                    out_hbm.at[c, pl.ds(r0, _ROWS_PER_TILE)])


@functools.lru_cache(maxsize=1)
def _build_segsum():
    return pl.kernel(
        _segsum_body,
        out_type=jax.ShapeDtypeStruct((_NC, _NPAD, _D), jnp.float32),
        mesh=plsc.VectorSubcoreMesh(core_axis_name="c", subcore_axis_name="s"),
        scratch_types=[
            pltpu.VMEM((_K,), jnp.int32),
            pltpu.VMEM((_K,), jnp.int32),
            pltpu.VMEM((_K, _D), jnp.float32),
            pltpu.VMEM_SHARED((_NPAD, _D), jnp.float32),
            pltpu.SemaphoreType.DMA,
        ],
    )


def _layer_body(h_ref, p_ref, w1_ref, b1_ref, w2_ref, b2_ref, g_ref, bb_ref,
                out_ref):
    z = h_ref[...] + p_ref[0, :_N] + p_ref[1, :_N]
    z1 = jnp.dot(z, w1_ref[...], preferred_element_type=jnp.float32,
                 precision=_HI) + b1_ref[...]
    z1 = jnp.maximum(z1, 0.0)
    u = jnp.dot(z1, w2_ref[...], preferred_element_type=jnp.float32,
                precision=_HI) + b2_ref[...]
    u = jnp.maximum(u, 0.0)
    mean = jnp.mean(u, axis=0, keepdims=True)
    var = jnp.mean((u - mean) * (u - mean), axis=0, keepdims=True)
    out_ref[...] = (g_ref[...] * (u - mean) * lax.rsqrt(var + 1e-5)
                    + bb_ref[...])


_layer_call = pl.pallas_call(
    _layer_body,
    out_shape=jax.ShapeDtypeStruct((_N, _D), jnp.float32),
)


def _sep_body(h_ref, batch_ref, w1_ref, b1_ref, g_ref, bb_ref, w2_ref, b2_ref,
              score_ref, pos_ref, neg_ref):
    s = jnp.dot(h_ref[...], w1_ref[...], preferred_element_type=jnp.float32,
                precision=_HI) + b1_ref[...]
    mean = jnp.mean(s, axis=0, keepdims=True)
    var = jnp.mean((s - mean) * (s - mean), axis=0, keepdims=True)
    s = g_ref[...] * (s - mean) * lax.rsqrt(var + 1e-5) + bb_ref[...]
    s = jnp.maximum(s, 0.0)
    logits = jnp.dot(s, w2_ref[...], preferred_element_type=jnp.float32,
                     precision=_HI) + b2_ref[...]
    score = jax.nn.sigmoid(logits)
    score_ref[...] = score
    pos_node = jnp.mean(score, axis=1, keepdims=True)
    gids = lax.broadcasted_iota(jnp.int32, (_N, _G), 1)
    mask = (batch_ref[...].reshape(_N, 1) == gids).astype(jnp.float32)
    pos_b = jnp.dot(pos_node.T, mask, preferred_element_type=jnp.float32,
                    precision=_HI)
    cnt_b = jnp.sum(mask, axis=0, keepdims=True)
    pos_ref[...] = pos_b + 1e-8
    neg_ref[...] = (cnt_b - pos_b) + 1e-8


_sep_call = pl.pallas_call(
    _sep_body,
    out_shape=(
        jax.ShapeDtypeStruct((_N, _D), jnp.float32),
        jax.ShapeDtypeStruct((1, _G), jnp.float32),
        jax.ShapeDtypeStruct((1, _G), jnp.float32),
    ),
)


def kernel(x, edge_index, batch, gin_W1, gin_b1, gin_W2, gin_b2, bn_g, bn_b,
           sep_W1, sep_b1, sep_bn_g, sep_bn_b, sep_W2, sep_b2):
    src = edge_index[0]
    dst = edge_index[1]
    zero = jnp.zeros((_NPAD, _D), jnp.float32)
    h = x
    segsum = _build_segsum()
    for i in range(_L):
        parts = segsum(h, src, dst, zero)
        h = _layer_call(h, parts,
                        gin_W1[i], gin_b1[i].reshape(1, _D),
                        gin_W2[i], gin_b2[i].reshape(1, _D),
                        bn_g[i].reshape(1, _D), bn_b[i].reshape(1, _D))
    score, pos_b, neg_b = _sep_call(
        h, batch, sep_W1, sep_b1.reshape(1, 2 * _D),
        sep_bn_g.reshape(1, 2 * _D), sep_bn_b.reshape(1, 2 * _D),
        sep_W2, sep_b2.reshape(1, _D))
    return score, pos_b.reshape(_G), neg_b.reshape(_G)


# speedup vs baseline: 8.0134x; 1.8915x over previous
import functools

import jax
import jax.numpy as jnp
from jax import lax
from jax.experimental import pallas as pl
from jax.experimental.pallas import tpu as pltpu
from jax.experimental.pallas import tpu_sc as plsc

_N = 10000
_E = 320000
_D = 128
_G = 128
_L = 5

_NC = 2
_NS = 16
_NW = _NC * _NS
_EPT = _E // _NW
_K = 80
_NCHUNK = _EPT // _K
_BC = 25
_NPAD = 10240
_ROWS_PER_TILE = _NPAD // _NS

_HI = jax.lax.Precision.HIGHEST


def _segsum_body(h_hbm, src_hbm, dst_hbm, zero_hbm, out_hbm,
                 src_v, dst_v, rows_a, rows_b, agg_sh,
                 sga, sgb, ssa, ssb):
    c = lax.axis_index("c")
    s = lax.axis_index("s")
    tid = c * _NS + s
    r0 = s * _ROWS_PER_TILE

    pltpu.async_copy(zero_hbm.at[pl.ds(r0, _ROWS_PER_TILE)],
                     agg_sh.at[pl.ds(r0, _ROWS_PER_TILE)], sga).wait()
    plsc.subcore_barrier()

    n = _BC

    def gather(j, buf, sem):
        pltpu.async_copy(h_hbm.at[src_v.at[j]], buf, sem)

    def scatter(j, buf, sem):
        pltpu.async_copy(buf, agg_sh.at[dst_v.at[j]], sem, add=True)

    def gwait(buf, sem):
        pltpu.make_async_copy(h_hbm.at[src_v.at[0]], buf, sem).wait()

    def swait(buf, sem):
        pltpu.make_async_copy(buf, agg_sh.at[dst_v.at[0]], sem).wait()

    def block(b, carry):
        pltpu.sync_copy(src_hbm.at[tid, b], src_v)
        pltpu.sync_copy(dst_hbm.at[tid, b], dst_v)

        gather(0, rows_a, sga)
        gwait(rows_a, sga)
        scatter(0, rows_a, ssa)
        gather(1, rows_b, sgb)

        def pair(t, c2):
            a = 2 * t - 1
            gwait(rows_b, sgb)
            scatter(a, rows_b, ssb)
            swait(rows_a, ssa)
            gather(a + 1, rows_a, sga)
            gwait(rows_a, sga)
            scatter(a + 1, rows_a, ssa)
            swait(rows_b, ssb)

            @pl.when(a + 2 < n)
            def _():
                gather(a + 2, rows_b, sgb)

            return c2

        lax.fori_loop(1, (n + 1) // 2, pair, 0)
        swait(rows_a, ssa)
        return carry

    lax.fori_loop(0, _NCHUNK // _BC, block, 0)

    plsc.subcore_barrier()
    pltpu.sync_copy(agg_sh.at[pl.ds(r0, _ROWS_PER_TILE)],
                    out_hbm.at[c, pl.ds(r0, _ROWS_PER_TILE)])


@functools.lru_cache(maxsize=1)
def _build_segsum():
    return pl.kernel(
        _segsum_body,
        out_type=jax.ShapeDtypeStruct((_NC, _NPAD, _D), jnp.float32),
        mesh=plsc.VectorSubcoreMesh(core_axis_name="c", subcore_axis_name="s"),
        scratch_types=[
            pltpu.VMEM((_BC, _K), jnp.int32),
            pltpu.VMEM((_BC, _K), jnp.int32),
            pltpu.VMEM((_K, _D), jnp.float32),
            pltpu.VMEM((_K, _D), jnp.float32),
            pltpu.VMEM_SHARED((_NPAD, _D), jnp.float32),
            pltpu.SemaphoreType.DMA,
            pltpu.SemaphoreType.DMA,
            pltpu.SemaphoreType.DMA,
            pltpu.SemaphoreType.DMA,
        ],
    )


def _layer_body(h_ref, p_ref, w1_ref, b1_ref, w2_ref, b2_ref, g_ref, bb_ref,
                out_ref):
    z = h_ref[...] + p_ref[0, :_N] + p_ref[1, :_N]
    z1 = jnp.dot(z, w1_ref[...],
                 preferred_element_type=jnp.float32) + b1_ref[...]
    z1 = jnp.maximum(z1, 0.0)
    u = jnp.dot(z1, w2_ref[...],
                preferred_element_type=jnp.float32) + b2_ref[...]
    u = jnp.maximum(u, 0.0)
    mean = jnp.mean(u, axis=0, keepdims=True)
    var = jnp.mean((u - mean) * (u - mean), axis=0, keepdims=True)
    out_ref[...] = (g_ref[...] * (u - mean) * lax.rsqrt(var + 1e-5)
                    + bb_ref[...])


_layer_call = pl.pallas_call(
    _layer_body,
    out_shape=jax.ShapeDtypeStruct((_N, _D), jnp.float32),
)


def _sep_body(h_ref, batch_ref, w1_ref, b1_ref, g_ref, bb_ref, w2_ref, b2_ref,
              score_ref, pos_ref, neg_ref):
    s = jnp.dot(h_ref[...], w1_ref[...],
                preferred_element_type=jnp.float32) + b1_ref[...]
    mean = jnp.mean(s, axis=0, keepdims=True)
    var = jnp.mean((s - mean) * (s - mean), axis=0, keepdims=True)
    s = g_ref[...] * (s - mean) * lax.rsqrt(var + 1e-5) + bb_ref[...]
    s = jnp.maximum(s, 0.0)
    logits = jnp.dot(s, w2_ref[...],
                     preferred_element_type=jnp.float32) + b2_ref[...]
    score = jax.nn.sigmoid(logits)
    score_ref[...] = score
    pos_node = jnp.mean(score, axis=1, keepdims=True)
    gids = lax.broadcasted_iota(jnp.int32, (_N, _G), 1)
    mask = (batch_ref[...].reshape(_N, 1) == gids).astype(jnp.float32)
    pos_b = jnp.dot(pos_node.T, mask, preferred_element_type=jnp.float32,
                    precision=_HI)
    cnt_b = jnp.sum(mask, axis=0, keepdims=True)
    pos_ref[...] = pos_b + 1e-8
    neg_ref[...] = (cnt_b - pos_b) + 1e-8


_sep_call = pl.pallas_call(
    _sep_body,
    out_shape=(
        jax.ShapeDtypeStruct((_N, _D), jnp.float32),
        jax.ShapeDtypeStruct((1, _G), jnp.float32),
        jax.ShapeDtypeStruct((1, _G), jnp.float32),
    ),
)


def kernel(x, edge_index, batch, gin_W1, gin_b1, gin_W2, gin_b2, bn_g, bn_b,
           sep_W1, sep_b1, sep_bn_g, sep_bn_b, sep_W2, sep_b2):
    src = edge_index[0].reshape(_NW, _NCHUNK // _BC, _BC, _K)
    dst = edge_index[1].reshape(_NW, _NCHUNK // _BC, _BC, _K)
    zero = jnp.zeros((_NPAD, _D), jnp.float32)
    h = x
    segsum = _build_segsum()
    for i in range(_L):
        parts = segsum(h, src, dst, zero)
        h = _layer_call(h, parts,
                        gin_W1[i], gin_b1[i].reshape(1, _D),
                        gin_W2[i], gin_b2[i].reshape(1, _D),
                        bn_g[i].reshape(1, _D), bn_b[i].reshape(1, _D))
    score, pos_b, neg_b = _sep_call(
        h, batch, sep_W1, sep_b1.reshape(1, 2 * _D),
        sep_bn_g.reshape(1, 2 * _D), sep_bn_b.reshape(1, 2 * _D),
        sep_W2, sep_b2.reshape(1, _D))
    return score, pos_b.reshape(_G), neg_b.reshape(_G)


# speedup vs baseline: 9.3142x; 1.1623x over previous
import functools

import jax
import jax.numpy as jnp
from jax import lax
from jax.experimental import pallas as pl
from jax.experimental.pallas import tpu as pltpu
from jax.experimental.pallas import tpu_sc as plsc

_N = 10000
_E = 320000
_D = 128
_G = 128
_L = 5

_NC = 2
_NS = 16
_NW = _NC * _NS
_K = 128
_NCHUNK = 80
_EPT = _NCHUNK * _K
_EP = _NW * _EPT
_BC = 20
_NPAD = 10240
_ROWS_PER_TILE = _NPAD // _NS

_HI = jax.lax.Precision.HIGHEST


def _segsum_body(h_hbm, src_hbm, dst_hbm, zero_hbm, out_hbm,
                 src_v, dst_v, rows_a, rows_b, agg_sh,
                 sga, sgb, ssa, ssb):
    c = lax.axis_index("c")
    s = lax.axis_index("s")
    tid = c * _NS + s
    r0 = s * _ROWS_PER_TILE

    pltpu.async_copy(zero_hbm.at[pl.ds(r0, _ROWS_PER_TILE)],
                     agg_sh.at[pl.ds(r0, _ROWS_PER_TILE)], sga).wait()
    plsc.subcore_barrier()

    n = _BC

    def gather(j, buf, sem):
        pltpu.async_copy(h_hbm.at[src_v.at[j]], buf, sem)

    def scatter(j, buf, sem):
        pltpu.async_copy(buf, agg_sh.at[dst_v.at[j]], sem, add=True)

    def gwait(buf, sem):
        pltpu.make_async_copy(h_hbm.at[src_v.at[0]], buf, sem).wait()

    def swait(buf, sem):
        pltpu.make_async_copy(buf, agg_sh.at[dst_v.at[0]], sem).wait()

    def block(b, carry):
        pltpu.sync_copy(src_hbm.at[tid, b], src_v)
        pltpu.sync_copy(dst_hbm.at[tid, b], dst_v)

        gather(0, rows_a, sga)
        gwait(rows_a, sga)
        scatter(0, rows_a, ssa)
        gather(1, rows_b, sgb)

        def pair(t, c2):
            a = 2 * t - 1
            gwait(rows_b, sgb)
            scatter(a, rows_b, ssb)
            swait(rows_a, ssa)
            gather(a + 1, rows_a, sga)
            gwait(rows_a, sga)
            scatter(a + 1, rows_a, ssa)
            swait(rows_b, ssb)

            @pl.when(a + 2 < n)
            def _():
                gather(a + 2, rows_b, sgb)

            return c2

        lax.fori_loop(1, (n + 1) // 2, pair, 0)
        if n % 2 == 0:
            gwait(rows_b, sgb)
            scatter(n - 1, rows_b, ssb)
            swait(rows_a, ssa)
            swait(rows_b, ssb)
        else:
            swait(rows_a, ssa)
        return carry

    lax.fori_loop(0, _NCHUNK // _BC, block, 0)

    plsc.subcore_barrier()
    pltpu.sync_copy(agg_sh.at[pl.ds(r0, _ROWS_PER_TILE)],
                    out_hbm.at[c, pl.ds(r0, _ROWS_PER_TILE)])


@functools.lru_cache(maxsize=1)
def _build_segsum():
    return pl.kernel(
        _segsum_body,
        out_type=jax.ShapeDtypeStruct((_NC, _NPAD, _D), jnp.float32),
        mesh=plsc.VectorSubcoreMesh(core_axis_name="c", subcore_axis_name="s"),
        scratch_types=[
            pltpu.VMEM((_BC, _K), jnp.int32),
            pltpu.VMEM((_BC, _K), jnp.int32),
            pltpu.VMEM((_K, _D), jnp.float32),
            pltpu.VMEM((_K, _D), jnp.float32),
            pltpu.VMEM_SHARED((_NPAD, _D), jnp.float32),
            pltpu.SemaphoreType.DMA,
            pltpu.SemaphoreType.DMA,
            pltpu.SemaphoreType.DMA,
            pltpu.SemaphoreType.DMA,
        ],
    )


def _layer_body(h_ref, p_ref, w1_ref, b1_ref, w2_ref, b2_ref, g_ref, bb_ref,
                out_ref):
    z = h_ref[...] + p_ref[0, :_N] + p_ref[1, :_N]
    z1 = jnp.dot(z, w1_ref[...],
                 preferred_element_type=jnp.float32) + b1_ref[...]
    z1 = jnp.maximum(z1, 0.0)
    u = jnp.dot(z1, w2_ref[...],
                preferred_element_type=jnp.float32) + b2_ref[...]
    u = jnp.maximum(u, 0.0)
    mean = jnp.mean(u, axis=0, keepdims=True)
    var = jnp.mean((u - mean) * (u - mean), axis=0, keepdims=True)
    out_ref[...] = (g_ref[...] * (u - mean) * lax.rsqrt(var + 1e-5)
                    + bb_ref[...])


_layer_call = pl.pallas_call(
    _layer_body,
    out_shape=jax.ShapeDtypeStruct((_N, _D), jnp.float32),
)


def _sep_body(h_ref, batch_ref, w1_ref, b1_ref, g_ref, bb_ref, w2_ref, b2_ref,
              score_ref, pos_ref, neg_ref):
    s = jnp.dot(h_ref[...], w1_ref[...],
                preferred_element_type=jnp.float32) + b1_ref[...]
    mean = jnp.mean(s, axis=0, keepdims=True)
    var = jnp.mean((s - mean) * (s - mean), axis=0, keepdims=True)
    s = g_ref[...] * (s - mean) * lax.rsqrt(var + 1e-5) + bb_ref[...]
    s = jnp.maximum(s, 0.0)
    logits = jnp.dot(s, w2_ref[...],
                     preferred_element_type=jnp.float32) + b2_ref[...]
    score = jax.nn.sigmoid(logits)
    score_ref[...] = score
    pos_node = jnp.mean(score, axis=1, keepdims=True)
    gids = lax.broadcasted_iota(jnp.int32, (_N, _G), 1)
    mask = (batch_ref[...].reshape(_N, 1) == gids).astype(jnp.float32)
    pos_b = jnp.dot(pos_node.T, mask, preferred_element_type=jnp.float32,
                    precision=_HI)
    cnt_b = jnp.sum(mask, axis=0, keepdims=True)
    pos_ref[...] = pos_b + 1e-8
    neg_ref[...] = (cnt_b - pos_b) + 1e-8


_sep_call = pl.pallas_call(
    _sep_body,
    out_shape=(
        jax.ShapeDtypeStruct((_N, _D), jnp.float32),
        jax.ShapeDtypeStruct((1, _G), jnp.float32),
        jax.ShapeDtypeStruct((1, _G), jnp.float32),
    ),
)


def kernel(x, edge_index, batch, gin_W1, gin_b1, gin_W2, gin_b2, bn_g, bn_b,
           sep_W1, sep_b1, sep_bn_g, sep_bn_b, sep_W2, sep_b2):
    npad = _EP - _E
    pad_src = (jnp.arange(npad, dtype=jnp.int32) * 13) % _N
    pad_dst = _N + (jnp.arange(npad, dtype=jnp.int32) % (_NPAD - _N))
    src = jnp.concatenate([edge_index[0], pad_src]).reshape(
        _NW, _NCHUNK // _BC, _BC, _K)
    dst = jnp.concatenate([edge_index[1], pad_dst]).reshape(
        _NW, _NCHUNK // _BC, _BC, _K)
    zero = jnp.zeros((_NPAD, _D), jnp.float32)
    h = x
    segsum = _build_segsum()
    for i in range(_L):
        parts = segsum(h, src, dst, zero)
        h = _layer_call(h, parts,
                        gin_W1[i], gin_b1[i].reshape(1, _D),
                        gin_W2[i], gin_b2[i].reshape(1, _D),
                        bn_g[i].reshape(1, _D), bn_b[i].reshape(1, _D))
    score, pos_b, neg_b = _sep_call(
        h, batch, sep_W1, sep_b1.reshape(1, 2 * _D),
        sep_bn_g.reshape(1, 2 * _D), sep_bn_b.reshape(1, 2 * _D),
        sep_W2, sep_b2.reshape(1, _D))
    return score, pos_b.reshape(_G), neg_b.reshape(_G)


# speedup vs baseline: 11.0099x; 1.1821x over previous
import functools

import jax
import jax.numpy as jnp
from jax import lax
from jax.experimental import pallas as pl
from jax.experimental.pallas import tpu as pltpu
from jax.experimental.pallas import tpu_sc as plsc

_N = 10000
_E = 320000
_D = 128
_G = 128
_L = 5

_NC = 2
_NS = 16
_NW = _NC * _NS
_K = 112
_NCHUNK = 92
_EPT = _NCHUNK * _K
_EP = _NW * _EPT
_BC = 23
_NPAD = 10112
_ROWS_PER_TILE = _NPAD // _NS

_HI = jax.lax.Precision.HIGHEST


def _segsum_body(h_hbm, src_hbm, dst_hbm, zero_hbm, out_hbm,
                 src_v, dst_v, r0b, r1b, r2b, agg_sh,
                 sg0, sg1, sg2, ss0, ss1, ss2):
    c = lax.axis_index("c")
    s = lax.axis_index("s")
    tid = c * _NS + s
    r0 = s * _ROWS_PER_TILE

    pltpu.async_copy(zero_hbm.at[pl.ds(r0, _ROWS_PER_TILE)],
                     agg_sh.at[pl.ds(r0, _ROWS_PER_TILE)], sg0).wait()
    plsc.subcore_barrier()

    n = _BC
    assert (n - 2) % 3 == 0

    def gather(j, buf, sem):
        pltpu.async_copy(h_hbm.at[src_v.at[j]], buf, sem)

    def scatter(j, buf, sem):
        pltpu.async_copy(buf, agg_sh.at[dst_v.at[j]], sem, add=True)

    def gwait(buf, sem):
        pltpu.make_async_copy(h_hbm.at[src_v.at[0]], buf, sem).wait()

    def swait(buf, sem):
        pltpu.make_async_copy(buf, agg_sh.at[dst_v.at[0]], sem).wait()

    B = ((r0b, sg0, ss0), (r1b, sg1, ss1), (r2b, sg2, ss2))

    def block(b, carry):
        pltpu.sync_copy(src_hbm.at[tid, b], src_v)
        pltpu.sync_copy(dst_hbm.at[tid, b], dst_v)

        gather(0, B[0][0], B[0][1])
        gather(1, B[1][0], B[1][1])
        gwait(B[0][0], B[0][1])
        scatter(0, B[0][0], B[0][2])
        gather(2, B[2][0], B[2][1])
        gwait(B[1][0], B[1][1])
        scatter(1, B[1][0], B[1][2])
        swait(B[0][0], B[0][2])
        gather(3, B[0][0], B[0][1])

        def step(cc, bx):
            buf, gs, ss = B[bx]
            pbuf, pgs, pss = B[(bx + 2) % 3]
            gwait(buf, gs)
            scatter(cc, buf, ss)
            swait(pbuf, pss)

            @pl.when(cc + 2 < n)
            def _():
                gather(cc + 2, pbuf, pgs)

        def trio(t, c2):
            cc = 3 * t + 2
            step(cc, 2)
            step(cc + 1, 0)
            step(cc + 2, 1)
            return c2

        lax.fori_loop(0, (n - 2) // 3, trio, 0)
        swait(B[(n - 1) % 3][0], B[(n - 1) % 3][2])
        return carry

    lax.fori_loop(0, _NCHUNK // _BC, block, 0)

    plsc.subcore_barrier()
    pltpu.sync_copy(agg_sh.at[pl.ds(r0, _ROWS_PER_TILE)],
                    out_hbm.at[c, pl.ds(r0, _ROWS_PER_TILE)])


@functools.lru_cache(maxsize=1)
def _build_segsum():
    return pl.kernel(
        _segsum_body,
        out_type=jax.ShapeDtypeStruct((_NC, _NPAD, _D), jnp.float32),
        mesh=plsc.VectorSubcoreMesh(core_axis_name="c", subcore_axis_name="s"),
        scratch_types=[
            pltpu.VMEM((_BC, _K), jnp.int32),
            pltpu.VMEM((_BC, _K), jnp.int32),
            pltpu.VMEM((_K, _D), jnp.float32),
            pltpu.VMEM((_K, _D), jnp.float32),
            pltpu.VMEM((_K, _D), jnp.float32),
            pltpu.VMEM_SHARED((_NPAD, _D), jnp.float32),
            pltpu.SemaphoreType.DMA,
            pltpu.SemaphoreType.DMA,
            pltpu.SemaphoreType.DMA,
            pltpu.SemaphoreType.DMA,
            pltpu.SemaphoreType.DMA,
            pltpu.SemaphoreType.DMA,
        ],
    )


def _layer_body(h_ref, p_ref, w1_ref, b1_ref, w2_ref, b2_ref, g_ref, bb_ref,
                out_ref):
    z = h_ref[...] + p_ref[0, :_N] + p_ref[1, :_N]
    z1 = jnp.dot(z, w1_ref[...],
                 preferred_element_type=jnp.float32) + b1_ref[...]
    z1 = jnp.maximum(z1, 0.0)
    u = jnp.dot(z1, w2_ref[...],
                preferred_element_type=jnp.float32) + b2_ref[...]
    u = jnp.maximum(u, 0.0)
    mean = jnp.mean(u, axis=0, keepdims=True)
    var = jnp.mean((u - mean) * (u - mean), axis=0, keepdims=True)
    out_ref[...] = (g_ref[...] * (u - mean) * lax.rsqrt(var + 1e-5)
                    + bb_ref[...])


_layer_call = pl.pallas_call(
    _layer_body,
    out_shape=jax.ShapeDtypeStruct((_N, _D), jnp.float32),
)


def _sep_body(h_ref, batch_ref, w1_ref, b1_ref, g_ref, bb_ref, w2_ref, b2_ref,
              score_ref, pos_ref, neg_ref):
    s = jnp.dot(h_ref[...], w1_ref[...],
                preferred_element_type=jnp.float32) + b1_ref[...]
    mean = jnp.mean(s, axis=0, keepdims=True)
    var = jnp.mean((s - mean) * (s - mean), axis=0, keepdims=True)
    s = g_ref[...] * (s - mean) * lax.rsqrt(var + 1e-5) + bb_ref[...]
    s = jnp.maximum(s, 0.0)
    logits = jnp.dot(s, w2_ref[...],
                     preferred_element_type=jnp.float32) + b2_ref[...]
    score = jax.nn.sigmoid(logits)
    score_ref[...] = score
    pos_node = jnp.mean(score, axis=1, keepdims=True)
    gids = lax.broadcasted_iota(jnp.int32, (_N, _G), 1)
    mask = (batch_ref[...].reshape(_N, 1) == gids).astype(jnp.float32)
    pos_b = jnp.dot(pos_node.T, mask, preferred_element_type=jnp.float32,
                    precision=_HI)
    cnt_b = jnp.sum(mask, axis=0, keepdims=True)
    pos_ref[...] = pos_b + 1e-8
    neg_ref[...] = (cnt_b - pos_b) + 1e-8


_sep_call = pl.pallas_call(
    _sep_body,
    out_shape=(
        jax.ShapeDtypeStruct((_N, _D), jnp.float32),
        jax.ShapeDtypeStruct((1, _G), jnp.float32),
        jax.ShapeDtypeStruct((1, _G), jnp.float32),
    ),
)


def kernel(x, edge_index, batch, gin_W1, gin_b1, gin_W2, gin_b2, bn_g, bn_b,
           sep_W1, sep_b1, sep_bn_g, sep_bn_b, sep_W2, sep_b2):
    npad = _EP - _E
    pad_src = (jnp.arange(npad, dtype=jnp.int32) * 13) % _N
    pad_dst = _N + (jnp.arange(npad, dtype=jnp.int32) % (_NPAD - _N))
    src = jnp.concatenate([edge_index[0], pad_src]).reshape(
        _NW, _NCHUNK // _BC, _BC, _K)
    dst = jnp.concatenate([edge_index[1], pad_dst]).reshape(
        _NW, _NCHUNK // _BC, _BC, _K)
    zero = jnp.zeros((_NPAD, _D), jnp.float32)
    h = x
    segsum = _build_segsum()
    for i in range(_L):
        parts = segsum(h, src, dst, zero)
        h = _layer_call(h, parts,
                        gin_W1[i], gin_b1[i].reshape(1, _D),
                        gin_W2[i], gin_b2[i].reshape(1, _D),
                        bn_g[i].reshape(1, _D), bn_b[i].reshape(1, _D))
    score, pos_b, neg_b = _sep_call(
        h, batch, sep_W1, sep_b1.reshape(1, 2 * _D),
        sep_bn_g.reshape(1, 2 * _D), sep_bn_b.reshape(1, 2 * _D),
        sep_W2, sep_b2.reshape(1, _D))
    return score, pos_b.reshape(_G), neg_b.reshape(_G)


# speedup vs baseline: 11.0384x; 1.0026x over previous
import functools

import jax
import jax.numpy as jnp
from jax import lax
from jax.experimental import pallas as pl
from jax.experimental.pallas import tpu as pltpu
from jax.experimental.pallas import tpu_sc as plsc

_N = 10000
_E = 320000
_D = 128
_G = 128
_L = 5

_NC = 2
_NS = 16
_NW = _NC * _NS
_K = 112
_NCHUNK = 92
_EPT = _NCHUNK * _K
_EP = _NW * _EPT
_BC = 23
_NPAD = 10112
_ROWS_PER_TILE = _NPAD // _NS

_HI = jax.lax.Precision.HIGHEST


def _segsum_body(h_hbm, src_hbm, dst_hbm, zero_hbm, out_hbm,
                 src_v, dst_v, r0b, r1b, r2b, agg_sh,
                 sg0, sg1, sg2, ss0, ss1, ss2):
    c = lax.axis_index("c")
    s = lax.axis_index("s")
    tid = c * _NS + s
    r0 = s * _ROWS_PER_TILE

    pltpu.async_copy(zero_hbm.at[pl.ds(r0, _ROWS_PER_TILE)],
                     agg_sh.at[pl.ds(r0, _ROWS_PER_TILE)], sg0).wait()
    plsc.subcore_barrier()

    n = _BC
    assert (n - 2) % 3 == 0

    def gather(j, buf, sem):
        pltpu.async_copy(h_hbm.at[src_v.at[j]], buf, sem)

    def scatter(j, buf, sem):
        pltpu.async_copy(buf, agg_sh.at[dst_v.at[j]], sem, add=True)

    def gwait(buf, sem):
        pltpu.make_async_copy(h_hbm.at[src_v.at[0]], buf, sem).wait()

    def swait(buf, sem):
        pltpu.make_async_copy(buf, agg_sh.at[dst_v.at[0]], sem).wait()

    B = ((r0b, sg0, ss0), (r1b, sg1, ss1), (r2b, sg2, ss2))

    def block(b, carry):
        pltpu.sync_copy(src_hbm.at[tid, b], src_v)
        pltpu.sync_copy(dst_hbm.at[tid, b], dst_v)

        gather(0, B[0][0], B[0][1])
        gather(1, B[1][0], B[1][1])
        gwait(B[0][0], B[0][1])
        scatter(0, B[0][0], B[0][2])
        gather(2, B[2][0], B[2][1])
        gwait(B[1][0], B[1][1])
        scatter(1, B[1][0], B[1][2])
        swait(B[0][0], B[0][2])
        gather(3, B[0][0], B[0][1])

        def step(cc, bx):
            buf, gs, ss = B[bx]
            pbuf, pgs, pss = B[(bx + 2) % 3]
            gwait(buf, gs)
            scatter(cc, buf, ss)
            swait(pbuf, pss)

            @pl.when(cc + 2 < n)
            def _():
                gather(cc + 2, pbuf, pgs)

        def trio(t, c2):
            cc = 3 * t + 2
            step(cc, 2)
            step(cc + 1, 0)
            step(cc + 2, 1)
            return c2

        lax.fori_loop(0, (n - 2) // 3, trio, 0)
        swait(B[(n - 1) % 3][0], B[(n - 1) % 3][2])
        return carry

    lax.fori_loop(0, _NCHUNK // _BC, block, 0)

    plsc.subcore_barrier()
    pltpu.sync_copy(agg_sh.at[pl.ds(r0, _ROWS_PER_TILE)],
                    out_hbm.at[c, pl.ds(r0, _ROWS_PER_TILE)])


@functools.lru_cache(maxsize=1)
def _build_segsum():
    return pl.kernel(
        _segsum_body,
        out_type=jax.ShapeDtypeStruct((_NC, _NPAD, _D), jnp.float32),
        mesh=plsc.VectorSubcoreMesh(core_axis_name="c", subcore_axis_name="s"),
        scratch_types=[
            pltpu.VMEM((_BC, _K), jnp.int32),
            pltpu.VMEM((_BC, _K), jnp.int32),
            pltpu.VMEM((_K, _D), jnp.float32),
            pltpu.VMEM((_K, _D), jnp.float32),
            pltpu.VMEM((_K, _D), jnp.float32),
            pltpu.VMEM_SHARED((_NPAD, _D), jnp.float32),
            pltpu.SemaphoreType.DMA,
            pltpu.SemaphoreType.DMA,
            pltpu.SemaphoreType.DMA,
            pltpu.SemaphoreType.DMA,
            pltpu.SemaphoreType.DMA,
            pltpu.SemaphoreType.DMA,
        ],
    )


def _layer_body(h_ref, p_ref, w1_ref, b1_ref, w2_ref, b2_ref, g_ref, bb_ref,
                out_ref):
    z = h_ref[...] + p_ref[0, :_N] + p_ref[1, :_N]
    z1 = jnp.dot(z, w1_ref[...],
                 preferred_element_type=jnp.float32) + b1_ref[...]
    z1 = jnp.maximum(z1, 0.0)
    u = jnp.dot(z1, w2_ref[...],
                preferred_element_type=jnp.float32) + b2_ref[...]
    u = jnp.maximum(u, 0.0)
    mean = jnp.mean(u, axis=0, keepdims=True)
    var = jnp.mean((u - mean) * (u - mean), axis=0, keepdims=True)
    out_ref[...] = (g_ref[...] * (u - mean) * lax.rsqrt(var + 1e-5)
                    + bb_ref[...])


_layer_call = pl.pallas_call(
    _layer_body,
    out_shape=jax.ShapeDtypeStruct((_N, _D), jnp.float32),
)


def _last_body(h_ref, p_ref, w1_ref, b1_ref, w2_ref, b2_ref, g_ref, bb_ref,
               batch_ref, sw1_ref, sb1_ref, sg_ref, sbb_ref, sw2_ref, sb2_ref,
               score_ref, pos_ref, neg_ref):
    z = h_ref[...] + p_ref[0, :_N] + p_ref[1, :_N]
    z1 = jnp.dot(z, w1_ref[...],
                 preferred_element_type=jnp.float32) + b1_ref[...]
    z1 = jnp.maximum(z1, 0.0)
    u = jnp.dot(z1, w2_ref[...],
                preferred_element_type=jnp.float32) + b2_ref[...]
    u = jnp.maximum(u, 0.0)
    mean = jnp.mean(u, axis=0, keepdims=True)
    var = jnp.mean((u - mean) * (u - mean), axis=0, keepdims=True)
    h = g_ref[...] * (u - mean) * lax.rsqrt(var + 1e-5) + bb_ref[...]

    s = jnp.dot(h, sw1_ref[...],
                preferred_element_type=jnp.float32) + sb1_ref[...]
    smean = jnp.mean(s, axis=0, keepdims=True)
    svar = jnp.mean((s - smean) * (s - smean), axis=0, keepdims=True)
    s = sg_ref[...] * (s - smean) * lax.rsqrt(svar + 1e-5) + sbb_ref[...]
    s = jnp.maximum(s, 0.0)
    logits = jnp.dot(s, sw2_ref[...],
                     preferred_element_type=jnp.float32) + sb2_ref[...]
    score = jax.nn.sigmoid(logits)
    score_ref[...] = score
    pos_node = jnp.mean(score, axis=1, keepdims=True)
    gids = lax.broadcasted_iota(jnp.int32, (_N, _G), 1)
    mask = (batch_ref[...].reshape(_N, 1) == gids).astype(jnp.float32)
    pos_b = jnp.dot(pos_node.T, mask, preferred_element_type=jnp.float32,
                    precision=_HI)
    cnt_b = jnp.sum(mask, axis=0, keepdims=True)
    pos_ref[...] = pos_b + 1e-8
    neg_ref[...] = (cnt_b - pos_b) + 1e-8


_last_call = pl.pallas_call(
    _last_body,
    out_shape=(
        jax.ShapeDtypeStruct((_N, _D), jnp.float32),
        jax.ShapeDtypeStruct((1, _G), jnp.float32),
        jax.ShapeDtypeStruct((1, _G), jnp.float32),
    ),
)


def _sep_body(h_ref, batch_ref, w1_ref, b1_ref, g_ref, bb_ref, w2_ref, b2_ref,
              score_ref, pos_ref, neg_ref):
    s = jnp.dot(h_ref[...], w1_ref[...],
                preferred_element_type=jnp.float32) + b1_ref[...]
    mean = jnp.mean(s, axis=0, keepdims=True)
    var = jnp.mean((s - mean) * (s - mean), axis=0, keepdims=True)
    s = g_ref[...] * (s - mean) * lax.rsqrt(var + 1e-5) + bb_ref[...]
    s = jnp.maximum(s, 0.0)
    logits = jnp.dot(s, w2_ref[...],
                     preferred_element_type=jnp.float32) + b2_ref[...]
    score = jax.nn.sigmoid(logits)
    score_ref[...] = score
    pos_node = jnp.mean(score, axis=1, keepdims=True)
    gids = lax.broadcasted_iota(jnp.int32, (_N, _G), 1)
    mask = (batch_ref[...].reshape(_N, 1) == gids).astype(jnp.float32)
    pos_b = jnp.dot(pos_node.T, mask, preferred_element_type=jnp.float32,
                    precision=_HI)
    cnt_b = jnp.sum(mask, axis=0, keepdims=True)
    pos_ref[...] = pos_b + 1e-8
    neg_ref[...] = (cnt_b - pos_b) + 1e-8


_sep_call = pl.pallas_call(
    _sep_body,
    out_shape=(
        jax.ShapeDtypeStruct((_N, _D), jnp.float32),
        jax.ShapeDtypeStruct((1, _G), jnp.float32),
        jax.ShapeDtypeStruct((1, _G), jnp.float32),
    ),
)


def kernel(x, edge_index, batch, gin_W1, gin_b1, gin_W2, gin_b2, bn_g, bn_b,
           sep_W1, sep_b1, sep_bn_g, sep_bn_b, sep_W2, sep_b2):
    npad = _EP - _E
    pad_src = (jnp.arange(npad, dtype=jnp.int32) * 13) % _N
    pad_dst = _N + (jnp.arange(npad, dtype=jnp.int32) % (_NPAD - _N))
    src = jnp.concatenate([edge_index[0], pad_src]).reshape(
        _NW, _NCHUNK // _BC, _BC, _K)
    dst = jnp.concatenate([edge_index[1], pad_dst]).reshape(
        _NW, _NCHUNK // _BC, _BC, _K)
    zero = jnp.zeros((_NPAD, _D), jnp.float32)
    h = x
    segsum = _build_segsum()
    for i in range(_L - 1):
        parts = segsum(h, src, dst, zero)
        h = _layer_call(h, parts,
                        gin_W1[i], gin_b1[i].reshape(1, _D),
                        gin_W2[i], gin_b2[i].reshape(1, _D),
                        bn_g[i].reshape(1, _D), bn_b[i].reshape(1, _D))
    i = _L - 1
    parts = segsum(h, src, dst, zero)
    score, pos_b, neg_b = _last_call(
        h, parts,
        gin_W1[i], gin_b1[i].reshape(1, _D),
        gin_W2[i], gin_b2[i].reshape(1, _D),
        bn_g[i].reshape(1, _D), bn_b[i].reshape(1, _D),
        batch, sep_W1, sep_b1.reshape(1, 2 * _D),
        sep_bn_g.reshape(1, 2 * _D), sep_bn_b.reshape(1, 2 * _D),
        sep_W2, sep_b2.reshape(1, _D))
    return score, pos_b.reshape(_G), neg_b.reshape(_G)
